# baseline (speedup 1.0000x reference)
import functools
import jax
import jax.numpy as jnp
from jax import lax
from jax.experimental import pallas as pl
from jax.experimental.pallas import tpu as pltpu
from jax.experimental.pallas import tpu_sc as plsc

_MESH = plsc.VectorSubcoreMesh(core_axis_name="c", subcore_axis_name="s")
_PARAMS = pltpu.CompilerParams(use_tc_tiling_on_sc=False,
                               needs_layout_passes=False)
_W = 400


def _tc1_body(x_ref, wgl_ref, w1_ref, a_ref, h_ref, tdst_ref, hn8_ref, u_ref,
              abt_ref):
    i = pl.program_id(0)
    x = x_ref[...]
    h = lax.dot(x, wgl_ref[...], preferred_element_type=jnp.float32)
    h_ref[...] = h
    tdst_ref[:, :64] = h
    tdst_ref[:, 64:] = lax.dot(x, w1_ref[...], preferred_element_type=jnp.float32)
    anorm = jnp.sqrt(jnp.sum(a_ref[...] ** 2))
    hn = anorm * jnp.sqrt(jnp.sum(h * h, axis=1, keepdims=True))
    hn8_ref[...] = jnp.broadcast_to(hn, hn8_ref.shape)
    bmax = jnp.max(hn).reshape(1, 1)
    u_ref[...] = jnp.where(i == 0, bmax, jnp.maximum(u_ref[...], bmax))
    abt_ref[...] = jnp.broadcast_to(a_ref[...], abt_ref.shape)


def _tc1(x, W_gl, W1, a, block_n=1000):
    n, d = x.shape
    return pl.pallas_call(
        _tc1_body,
        grid=(n // block_n,),
        in_specs=[
            pl.BlockSpec((block_n, d), lambda i: (i, 0)),
            pl.BlockSpec((d, 64), lambda i: (0, 0)),
            pl.BlockSpec((d, 32), lambda i: (0, 0)),
            pl.BlockSpec((64, 1), lambda i: (0, 0)),
        ],
        out_specs=[
            pl.BlockSpec((block_n, 64), lambda i: (i, 0)),
            pl.BlockSpec((block_n, 96), lambda i: (i, 0)),
            pl.BlockSpec((block_n, 8), lambda i: (i, 0)),
            pl.BlockSpec((1, 1), lambda i: (0, 0)),
            pl.BlockSpec((64, 16), lambda i: (0, 0)),
        ],
        out_shape=[
            jax.ShapeDtypeStruct((n, 64), jnp.float32),
            jax.ShapeDtypeStruct((n, 96), jnp.float32),
            jax.ShapeDtypeStruct((n, 8), jnp.float32),
            jax.ShapeDtypeStruct((1, 1), jnp.float32),
            jax.ShapeDtypeStruct((64, 16), jnp.float32),
        ],
    )(x, W_gl, W1, a)


def _tc1b_body(h_ref, hn8_ref, u_ref, tsrc_ref):
    tsrc_ref[:, :64] = h_ref[...]
    tsrc_ref[:, 64:] = hn8_ref[...] + u_ref[0, 0]


def _tc1b(h, hn8, U, block_n=1000):
    n = h.shape[0]
    return pl.pallas_call(
        _tc1b_body,
        grid=(n // block_n,),
        in_specs=[
            pl.BlockSpec((block_n, 64), lambda i: (i, 0)),
            pl.BlockSpec((block_n, 8), lambda i: (i, 0)),
            pl.BlockSpec((1, 1), lambda i: (0, 0)),
        ],
        out_specs=pl.BlockSpec((block_n, 72), lambda i: (i, 0)),
        out_shape=jax.ShapeDtypeStruct((n, 72), jnp.float32),
    )(h, hn8, U)


def _sc1(tsrc, tdst, src2, dst2, abt, n, e):
    @functools.partial(
        pl.kernel,
        out_type=(
            jax.ShapeDtypeStruct((1, e), jnp.float32),
            jax.ShapeDtypeStruct((2, n, 40), jnp.float32),
        ),
        mesh=_MESH,
        scratch_types=[
            pltpu.VMEM((_W, 72), jnp.float32),
            pltpu.VMEM((_W, 96), jnp.float32),
            pltpu.VMEM((_W, 40), jnp.float32),
            pltpu.VMEM((64, 16), jnp.float32),
            pltpu.VMEM_SHARED((n, 40), jnp.float32),
        ],
        compiler_params=_PARAMS,
    )
    def k(tsrc_hbm, tdst_hbm, src_hbm, dst_hbm, abt_hbm, z_hbm,
          ex_hbm, acc_hbm, gs_v, gd_v, p_v, abt_v, sh):
        c = lax.axis_index("c")
        s = lax.axis_index("s")
        pltpu.sync_copy(abt_hbm, abt_v)

        @pl.when(s == 0)
        def _():
            pltpu.sync_copy(z_hbm, sh)

        plsc.subcore_barrier()

        rows0 = lax.iota(jnp.int32, 16)

        def body(src_v, dst_v, ex_v):
            pltpu.sync_copy(tsrc_hbm.at[src_v.at[0]], gs_v)
            pltpu.sync_copy(tdst_hbm.at[dst_v.at[0]], gd_v)

            @pl.loop(0, _W // 16)
            def _(g):
                rows = rows0 + g * 16
                accs = [jnp.zeros((16,), jnp.float32) for _ in range(4)]
                for kk in range(64):
                    ck = jnp.full((16,), kk, jnp.int32)
                    vs = plsc.load_gather(gs_v, [rows, ck])
                    vd = plsc.load_gather(gd_v, [rows, ck])
                    accs[kk % 4] = accs[kk % 4] + jnp.abs(vs - vd) * abt_v[kk, :]
                acc = (accs[0] + accs[1]) + (accs[2] + accs[3])
                hnu = plsc.load_gather(gs_v, [rows, jnp.full((16,), 64, jnp.int32)])
                ex = jnp.maximum(jnp.exp(jnp.maximum(acc, 0.0) - hnu), 1e-30)
                ex_v[0, pl.ds(g * 16, 16)] = ex
                for kk in range(32):
                    col = plsc.load_gather(
                        gd_v, [rows, jnp.full((16,), 64 + kk, jnp.int32)])
                    plsc.store_scatter(
                        p_v, [rows, jnp.full((16,), kk, jnp.int32)], ex * col)
                plsc.store_scatter(
                    p_v, [rows, jnp.full((16,), 32, jnp.int32)], ex)

            pltpu.sync_copy(p_v, sh.at[src_v.at[0]], add=True)

        pltpu.emit_pipeline(
            body,
            grid=(e // _W,),
            in_specs=[
                pl.BlockSpec((1, _W), lambda i: (0, i)),
                pl.BlockSpec((1, _W), lambda i: (0, i)),
            ],
            out_specs=[pl.BlockSpec((1, _W), lambda i: (0, i))],
            core_axis_name=("c", "s"),
            dimension_semantics=(pltpu.PARALLEL,),
        )(src_hbm, dst_hbm, ex_hbm)

        plsc.subcore_barrier()

        @pl.when(s == 0)
        def _():
            pltpu.sync_copy(sh, acc_hbm.at[c])

    z = jnp.zeros((n, 40), jnp.float32)
    return k(tsrc, tdst, src2, dst2, abt, z)


def _tc3_body(acc_ref, w2_ref, xw2_ref, rs8_ref):
    tot = acc_ref[0] + acc_ref[1]
    rs = tot[:, 32:33]
    x1 = jax.nn.relu(jnp.where(rs > 0, tot[:, :32] / rs, 0.0))
    xw2_ref[...] = lax.dot(x1, w2_ref[...], preferred_element_type=jnp.float32)
    rs8_ref[...] = jnp.broadcast_to(rs, rs8_ref.shape)


def _tc3(acc, W2, block_n=1000):
    n = acc.shape[1]
    return pl.pallas_call(
        _tc3_body,
        grid=(n // block_n,),
        in_specs=[
            pl.BlockSpec((2, block_n, 40), lambda i: (0, i, 0)),
            pl.BlockSpec((32, 16), lambda i: (0, 0)),
        ],
        out_specs=[
            pl.BlockSpec((block_n, 16), lambda i: (i, 0)),
            pl.BlockSpec((block_n, 8), lambda i: (i, 0)),
        ],
        out_shape=[
            jax.ShapeDtypeStruct((n, 16), jnp.float32),
            jax.ShapeDtypeStruct((n, 8), jnp.float32),
        ],
    )(acc, W2)


def _sc2(xw2, rs8, ex, src2, dst2, n, e):
    @functools.partial(
        pl.kernel,
        out_type=(
            jax.ShapeDtypeStruct((1, e), jnp.float32),
            jax.ShapeDtypeStruct((2, n, 16), jnp.float32),
        ),
        mesh=_MESH,
        scratch_types=[
            pltpu.VMEM((_W, 16), jnp.float32),
            pltpu.VMEM((_W, 8), jnp.float32),
            pltpu.VMEM((_W, 16), jnp.float32),
            pltpu.VMEM_SHARED((n, 16), jnp.float32),
        ],
        compiler_params=_PARAMS,
    )
    def k(xw2_hbm, rs8_hbm, ex_hbm, src_hbm, dst_hbm, z_hbm,
          adj_hbm, out_hbm, g2_v, rs_v, p2_v, sh):
        c = lax.axis_index("c")
        s = lax.axis_index("s")

        @pl.when(s == 0)
        def _():
            pltpu.sync_copy(z_hbm, sh)

        plsc.subcore_barrier()

        rows0 = lax.iota(jnp.int32, 16)

        def body(ex_v, src_v, dst_v, adj_v):
            pltpu.sync_copy(xw2_hbm.at[dst_v.at[0]], g2_v)
            pltpu.sync_copy(rs8_hbm.at[src_v.at[0]], rs_v)

            @pl.loop(0, _W // 16)
            def _(g):
                rows = rows0 + g * 16
                exv = ex_v[0, pl.ds(g * 16, 16)]
                rsv = plsc.load_gather(rs_v, [rows, jnp.full((16,), 0, jnp.int32)])
                adj = exv / rsv
                adj_v[0, pl.ds(g * 16, 16)] = adj
                for kk in range(16):
                    col = plsc.load_gather(
                        g2_v, [rows, jnp.full((16,), kk, jnp.int32)])
                    plsc.store_scatter(
                        p2_v, [rows, jnp.full((16,), kk, jnp.int32)], adj * col)

            pltpu.sync_copy(p2_v, sh.at[src_v.at[0]], add=True)

        pltpu.emit_pipeline(
            body,
            grid=(e // _W,),
            in_specs=[
                pl.BlockSpec((1, _W), lambda i: (0, i)),
                pl.BlockSpec((1, _W), lambda i: (0, i)),
                pl.BlockSpec((1, _W), lambda i: (0, i)),
            ],
            out_specs=[pl.BlockSpec((1, _W), lambda i: (0, i))],
            core_axis_name=("c", "s"),
            dimension_semantics=(pltpu.PARALLEL,),
        )(ex_hbm, src_hbm, dst_hbm, adj_hbm)

        plsc.subcore_barrier()

        @pl.when(s == 0)
        def _():
            pltpu.sync_copy(sh, out_hbm.at[c])

    z = jnp.zeros((n, 16), jnp.float32)
    return k(xw2, rs8, ex, src2, dst2, z)


def _tc5_body(p_ref, o_ref):
    o_ref[...] = p_ref[0] + p_ref[1]


def _tc5(parts, block_n=1000):
    n, d = parts.shape[1], parts.shape[2]
    return pl.pallas_call(
        _tc5_body,
        grid=(n // block_n,),
        in_specs=[pl.BlockSpec((2, block_n, d), lambda i: (0, i, 0))],
        out_specs=pl.BlockSpec((block_n, d), lambda i: (i, 0)),
        out_shape=jax.ShapeDtypeStruct((n, d), jnp.float32),
    )(parts)


def kernel(x, edge, num_nodes, W_gl, a, W1, W2):
    n = x.shape[0]
    e = edge.shape[1]
    src2 = edge[0:1]
    dst2 = edge[1:2]

    h, tdst, hn8, U, abt = _tc1(x, W_gl, W1, a)
    tsrc = _tc1b(h, hn8, U)
    ex, acc = _sc1(tsrc, tdst, src2, dst2, abt, n, e)
    xw2, rs8 = _tc3(acc, W2)
    adj, parts = _sc2(xw2, rs8, ex, src2, dst2, n, e)
    output = _tc5(parts)
    return (output, adj[0], h)


# speedup vs baseline: 1.0914x; 1.0914x over previous
import functools
import jax
import jax.numpy as jnp
from jax import lax
from jax.experimental import pallas as pl
from jax.experimental.pallas import tpu as pltpu
from jax.experimental.pallas import tpu_sc as plsc

_MESH = plsc.VectorSubcoreMesh(core_axis_name="c", subcore_axis_name="s")
_PARAMS = pltpu.CompilerParams(use_tc_tiling_on_sc=False,
                               needs_layout_passes=False)
_W = 400


def _tc1_body(x_ref, wgl_ref, w1_ref, a_ref, h_ref, tdst_ref, hn8_ref, u_ref,
              abt_ref):
    i = pl.program_id(0)
    x = x_ref[...]
    h = lax.dot(x, wgl_ref[...], preferred_element_type=jnp.float32)
    h_ref[...] = h
    tdst_ref[:, :64] = h
    tdst_ref[:, 64:] = lax.dot(x, w1_ref[...], preferred_element_type=jnp.float32)
    anorm = jnp.sqrt(jnp.sum(a_ref[...] ** 2))
    hn = anorm * jnp.sqrt(jnp.sum(h * h, axis=1, keepdims=True))
    hn8_ref[...] = jnp.broadcast_to(hn, hn8_ref.shape)
    bmax = jnp.max(hn).reshape(1, 1)
    u_ref[...] = jnp.where(i == 0, bmax, jnp.maximum(u_ref[...], bmax))
    abt_ref[...] = jnp.broadcast_to(a_ref[...], abt_ref.shape)


def _tc1(x, W_gl, W1, a, block_n=1000):
    n, d = x.shape
    return pl.pallas_call(
        _tc1_body,
        grid=(n // block_n,),
        in_specs=[
            pl.BlockSpec((block_n, d), lambda i: (i, 0)),
            pl.BlockSpec((d, 64), lambda i: (0, 0)),
            pl.BlockSpec((d, 32), lambda i: (0, 0)),
            pl.BlockSpec((64, 1), lambda i: (0, 0)),
        ],
        out_specs=[
            pl.BlockSpec((block_n, 64), lambda i: (i, 0)),
            pl.BlockSpec((block_n, 96), lambda i: (i, 0)),
            pl.BlockSpec((block_n, 8), lambda i: (i, 0)),
            pl.BlockSpec((1, 1), lambda i: (0, 0)),
            pl.BlockSpec((64, 16), lambda i: (0, 0)),
        ],
        out_shape=[
            jax.ShapeDtypeStruct((n, 64), jnp.float32),
            jax.ShapeDtypeStruct((n, 96), jnp.float32),
            jax.ShapeDtypeStruct((n, 8), jnp.float32),
            jax.ShapeDtypeStruct((1, 1), jnp.float32),
            jax.ShapeDtypeStruct((64, 16), jnp.float32),
        ],
    )(x, W_gl, W1, a)


def _tc1b_body(h_ref, hn8_ref, u_ref, tsrc_ref):
    tsrc_ref[:, :64] = h_ref[...]
    tsrc_ref[:, 64:] = hn8_ref[...] + u_ref[0, 0]


def _tc1b(h, hn8, U, block_n=1000):
    n = h.shape[0]
    return pl.pallas_call(
        _tc1b_body,
        grid=(n // block_n,),
        in_specs=[
            pl.BlockSpec((block_n, 64), lambda i: (i, 0)),
            pl.BlockSpec((block_n, 8), lambda i: (i, 0)),
            pl.BlockSpec((1, 1), lambda i: (0, 0)),
        ],
        out_specs=pl.BlockSpec((block_n, 72), lambda i: (i, 0)),
        out_shape=jax.ShapeDtypeStruct((n, 72), jnp.float32),
    )(h, hn8, U)


_C = 80


def _sc1(tsrc, tdst, src3, dst3, abt, n, e):
    ew = e // 32
    nch = ew // _C

    @functools.partial(
        pl.kernel,
        out_type=(
            jax.ShapeDtypeStruct((e,), jnp.float32),
            jax.ShapeDtypeStruct((2, n, 40), jnp.float32),
        ),
        mesh=_MESH,
        scratch_types=[
            pltpu.VMEM((2, _C, 72), jnp.float32),
            pltpu.VMEM((2, _C, 96), jnp.float32),
            pltpu.VMEM((_C, 40), jnp.float32),
            pltpu.VMEM((_C,), jnp.float32),
            pltpu.VMEM((nch, _C), jnp.int32),
            pltpu.VMEM((nch, _C), jnp.int32),
            pltpu.VMEM((64, 16), jnp.float32),
            pltpu.VMEM_SHARED((n, 40), jnp.float32),
            pltpu.SemaphoreType.DMA,
            pltpu.SemaphoreType.DMA,
            pltpu.SemaphoreType.DMA,
            pltpu.SemaphoreType.DMA,
        ],
        compiler_params=_PARAMS,
    )
    def k(tsrc_hbm, tdst_hbm, src_hbm, dst_hbm, abt_hbm, z_hbm,
          ex_hbm, acc_hbm, gs_v, gd_v, p_v, exb_v, srcv, dstv, abt_v, sh,
          ss0, ss1, sd0, sd1):
        c = lax.axis_index("c")
        s = lax.axis_index("s")
        wid = s * 2 + c
        pltpu.sync_copy(src_hbm.at[wid], srcv)
        pltpu.sync_copy(dst_hbm.at[wid], dstv)
        pltpu.sync_copy(abt_hbm, abt_v)

        @pl.when(s == 0)
        def _():
            pltpu.sync_copy(z_hbm, sh)

        plsc.subcore_barrier()

        sems_s = (ss0, ss1)
        sems_d = (sd0, sd1)

        def gather_pair(g, par):
            return (
                pltpu.make_async_copy(
                    tsrc_hbm.at[srcv.at[g]], gs_v.at[par], sems_s[par]),
                pltpu.make_async_copy(
                    tdst_hbm.at[dstv.at[g]], gd_v.at[par], sems_d[par]),
            )

        a0, b0 = gather_pair(0, 0)
        a0.start()
        b0.start()

        rows0 = lax.iota(jnp.int32, 16)
        base_w = wid * ew

        def process(g, buf):
            aw, bw = gather_pair(g, buf)
            aw.wait()
            bw.wait()

            @pl.when(g + 1 < nch)
            def _():
                an, bn = gather_pair(g + 1, 1 - buf)
                an.start()
                bn.start()

            gs = gs_v.at[buf]
            gd = gd_v.at[buf]

            @pl.loop(0, _C // 16)
            def _(gr):
                rows = rows0 + gr * 16
                accs = [jnp.zeros((16,), jnp.float32) for _ in range(4)]
                for kk in range(64):
                    ck = jnp.full((16,), kk, jnp.int32)
                    vs = plsc.load_gather(gs, [rows, ck])
                    vd = plsc.load_gather(gd, [rows, ck])
                    accs[kk % 4] = accs[kk % 4] + jnp.abs(vs - vd) * abt_v[kk, :]
                acc = (accs[0] + accs[1]) + (accs[2] + accs[3])
                hnu = plsc.load_gather(
                    gs, [rows, jnp.full((16,), 64, jnp.int32)])
                ex = jnp.maximum(jnp.exp(jnp.maximum(acc, 0.0) - hnu), 1e-30)
                exb_v[pl.ds(gr * 16, 16)] = ex
                for kk in range(32):
                    col = plsc.load_gather(
                        gd, [rows, jnp.full((16,), 64 + kk, jnp.int32)])
                    plsc.store_scatter(
                        p_v, [rows, jnp.full((16,), kk, jnp.int32)], ex * col)
                plsc.store_scatter(
                    p_v, [rows, jnp.full((16,), 32, jnp.int32)], ex)

            pltpu.sync_copy(p_v, sh.at[srcv.at[g]], add=True)
            pltpu.sync_copy(exb_v, ex_hbm.at[pl.ds(base_w + g * _C, _C)])

        process(0, 0)

        @pl.loop(0, (nch - 1) // 2)
        def _(t):
            for par in (0, 1):
                process(1 + t * 2 + par, 1 - par)

        plsc.subcore_barrier()

        @pl.when(s == 0)
        def _():
            pltpu.sync_copy(sh, acc_hbm.at[c])

    z = jnp.zeros((n, 40), jnp.float32)
    return k(tsrc, tdst, src3, dst3, abt, z)


def _tc3_body(acc_ref, w2_ref, xw2_ref, rs8_ref):
    tot = acc_ref[0] + acc_ref[1]
    rs = tot[:, 32:33]
    x1 = jax.nn.relu(jnp.where(rs > 0, tot[:, :32] / rs, 0.0))
    xw2_ref[...] = lax.dot(x1, w2_ref[...], preferred_element_type=jnp.float32)
    rs8_ref[...] = jnp.broadcast_to(rs, rs8_ref.shape)


def _tc3(acc, W2, block_n=1000):
    n = acc.shape[1]
    return pl.pallas_call(
        _tc3_body,
        grid=(n // block_n,),
        in_specs=[
            pl.BlockSpec((2, block_n, 40), lambda i: (0, i, 0)),
            pl.BlockSpec((32, 16), lambda i: (0, 0)),
        ],
        out_specs=[
            pl.BlockSpec((block_n, 16), lambda i: (i, 0)),
            pl.BlockSpec((block_n, 8), lambda i: (i, 0)),
        ],
        out_shape=[
            jax.ShapeDtypeStruct((n, 16), jnp.float32),
            jax.ShapeDtypeStruct((n, 8), jnp.float32),
        ],
    )(acc, W2)


def _sc2(xw2, rs8, ex, src2, dst2, n, e):
    @functools.partial(
        pl.kernel,
        out_type=(
            jax.ShapeDtypeStruct((1, e), jnp.float32),
            jax.ShapeDtypeStruct((2, n, 16), jnp.float32),
        ),
        mesh=_MESH,
        scratch_types=[
            pltpu.VMEM((_W, 16), jnp.float32),
            pltpu.VMEM((_W, 8), jnp.float32),
            pltpu.VMEM((_W, 16), jnp.float32),
            pltpu.VMEM_SHARED((n, 16), jnp.float32),
        ],
        compiler_params=_PARAMS,
    )
    def k(xw2_hbm, rs8_hbm, ex_hbm, src_hbm, dst_hbm, z_hbm,
          adj_hbm, out_hbm, g2_v, rs_v, p2_v, sh):
        c = lax.axis_index("c")
        s = lax.axis_index("s")

        @pl.when(s == 0)
        def _():
            pltpu.sync_copy(z_hbm, sh)

        plsc.subcore_barrier()

        rows0 = lax.iota(jnp.int32, 16)

        def body(ex_v, src_v, dst_v, adj_v):
            pltpu.sync_copy(xw2_hbm.at[dst_v.at[0]], g2_v)
            pltpu.sync_copy(rs8_hbm.at[src_v.at[0]], rs_v)

            @pl.loop(0, _W // 16)
            def _(g):
                rows = rows0 + g * 16
                exv = ex_v[0, pl.ds(g * 16, 16)]
                rsv = plsc.load_gather(rs_v, [rows, jnp.full((16,), 0, jnp.int32)])
                adj = exv / rsv
                adj_v[0, pl.ds(g * 16, 16)] = adj
                for kk in range(16):
                    col = plsc.load_gather(
                        g2_v, [rows, jnp.full((16,), kk, jnp.int32)])
                    plsc.store_scatter(
                        p2_v, [rows, jnp.full((16,), kk, jnp.int32)], adj * col)

            pltpu.sync_copy(p2_v, sh.at[src_v.at[0]], add=True)

        pltpu.emit_pipeline(
            body,
            grid=(e // _W,),
            in_specs=[
                pl.BlockSpec((1, _W), lambda i: (0, i)),
                pl.BlockSpec((1, _W), lambda i: (0, i)),
                pl.BlockSpec((1, _W), lambda i: (0, i)),
            ],
            out_specs=[pl.BlockSpec((1, _W), lambda i: (0, i))],
            core_axis_name=("c", "s"),
            dimension_semantics=(pltpu.PARALLEL,),
        )(ex_hbm, src_hbm, dst_hbm, adj_hbm)

        plsc.subcore_barrier()

        @pl.when(s == 0)
        def _():
            pltpu.sync_copy(sh, out_hbm.at[c])

    z = jnp.zeros((n, 16), jnp.float32)
    return k(xw2, rs8, ex, src2, dst2, z)


def _tc5_body(p_ref, o_ref):
    o_ref[...] = p_ref[0] + p_ref[1]


def _tc5(parts, block_n=1000):
    n, d = parts.shape[1], parts.shape[2]
    return pl.pallas_call(
        _tc5_body,
        grid=(n // block_n,),
        in_specs=[pl.BlockSpec((2, block_n, d), lambda i: (0, i, 0))],
        out_specs=pl.BlockSpec((block_n, d), lambda i: (i, 0)),
        out_shape=jax.ShapeDtypeStruct((n, d), jnp.float32),
    )(parts)


def kernel(x, edge, num_nodes, W_gl, a, W1, W2):
    n = x.shape[0]
    e = edge.shape[1]
    src2 = edge[0:1]
    dst2 = edge[1:2]
    ew = e // 32
    src3 = edge[0].reshape(32, ew // _C, _C)
    dst3 = edge[1].reshape(32, ew // _C, _C)

    h, tdst, hn8, U, abt = _tc1(x, W_gl, W1, a)
    tsrc = _tc1b(h, hn8, U)
    ex, acc = _sc1(tsrc, tdst, src3, dst3, abt, n, e)
    xw2, rs8 = _tc3(acc, W2)
    adj, parts = _sc2(xw2, rs8, ex.reshape(1, e), src2, dst2, n, e)
    output = _tc5(parts)
    return (output, adj[0], h)


# speedup vs baseline: 2.1056x; 1.9293x over previous
import functools
import jax
import jax.numpy as jnp
from jax import lax
from jax.experimental import pallas as pl
from jax.experimental.pallas import tpu as pltpu
from jax.experimental.pallas import tpu_sc as plsc

_MESH = plsc.VectorSubcoreMesh(core_axis_name="c", subcore_axis_name="s")
_PARAMS = pltpu.CompilerParams(use_tc_tiling_on_sc=False,
                               needs_layout_passes=False)
_W = 400


def _tc1_body(x_ref, wgl_ref, w1_ref, a_ref, h_ref, tdst_ref, hn8_ref, u_ref,
              abt_ref):
    i = pl.program_id(0)
    x = x_ref[...]
    h = lax.dot(x, wgl_ref[...], preferred_element_type=jnp.float32)
    h_ref[...] = h
    tdst_ref[:, :64] = h
    tdst_ref[:, 64:] = lax.dot(x, w1_ref[...], preferred_element_type=jnp.float32)
    anorm = jnp.sqrt(jnp.sum(a_ref[...] ** 2))
    hn = anorm * jnp.sqrt(jnp.sum(h * h, axis=1, keepdims=True))
    hn8_ref[...] = jnp.broadcast_to(hn, hn8_ref.shape)
    bmax = jnp.max(hn).reshape(1, 1)
    u_ref[...] = jnp.where(i == 0, bmax, jnp.maximum(u_ref[...], bmax))
    abt_ref[...] = jnp.reshape(a_ref[...], abt_ref.shape)


def _tc1(x, W_gl, W1, a, block_n=1000):
    n, d = x.shape
    return pl.pallas_call(
        _tc1_body,
        grid=(n // block_n,),
        in_specs=[
            pl.BlockSpec((block_n, d), lambda i: (i, 0)),
            pl.BlockSpec((d, 64), lambda i: (0, 0)),
            pl.BlockSpec((d, 32), lambda i: (0, 0)),
            pl.BlockSpec((64, 1), lambda i: (0, 0)),
        ],
        out_specs=[
            pl.BlockSpec((block_n, 64), lambda i: (i, 0)),
            pl.BlockSpec((block_n, 96), lambda i: (i, 0)),
            pl.BlockSpec((block_n, 8), lambda i: (i, 0)),
            pl.BlockSpec((1, 1), lambda i: (0, 0)),
            pl.BlockSpec((4, 16), lambda i: (0, 0)),
        ],
        out_shape=[
            jax.ShapeDtypeStruct((n, 64), jnp.float32),
            jax.ShapeDtypeStruct((n, 96), jnp.float32),
            jax.ShapeDtypeStruct((n, 8), jnp.float32),
            jax.ShapeDtypeStruct((1, 1), jnp.float32),
            jax.ShapeDtypeStruct((4, 16), jnp.float32),
        ],
    )(x, W_gl, W1, a)


def _tc1b_body(h_ref, hn8_ref, u_ref, tsrc_ref):
    tsrc_ref[:, :64] = h_ref[...]
    tsrc_ref[:, 64:] = hn8_ref[...] + u_ref[0, 0]


def _tc1b(h, hn8, U, block_n=1000):
    n = h.shape[0]
    return pl.pallas_call(
        _tc1b_body,
        grid=(n // block_n,),
        in_specs=[
            pl.BlockSpec((block_n, 64), lambda i: (i, 0)),
            pl.BlockSpec((block_n, 8), lambda i: (i, 0)),
            pl.BlockSpec((1, 1), lambda i: (0, 0)),
        ],
        out_specs=pl.BlockSpec((block_n, 72), lambda i: (i, 0)),
        out_shape=jax.ShapeDtypeStruct((n, 72), jnp.float32),
    )(h, hn8, U)


_C = 80


def _sc1(tsrc, tdst, src3, dst3, abt, n, e):
    ew = e // 32
    nch = ew // _C

    @functools.partial(
        pl.kernel,
        out_type=(
            jax.ShapeDtypeStruct((e,), jnp.float32),
            jax.ShapeDtypeStruct((2, n, 40), jnp.float32),
        ),
        mesh=_MESH,
        scratch_types=[
            pltpu.VMEM((2, _C, 72), jnp.float32),
            pltpu.VMEM((2, _C, 96), jnp.float32),
            pltpu.VMEM((_C, 40), jnp.float32),
            pltpu.VMEM((_C,), jnp.float32),
            pltpu.VMEM((nch, _C), jnp.int32),
            pltpu.VMEM((nch, _C), jnp.int32),
            pltpu.VMEM((4, 16), jnp.float32),
            pltpu.VMEM_SHARED((n, 40), jnp.float32),
            pltpu.SemaphoreType.DMA,
            pltpu.SemaphoreType.DMA,
            pltpu.SemaphoreType.DMA,
            pltpu.SemaphoreType.DMA,
        ],
        compiler_params=_PARAMS,
    )
    def k(tsrc_hbm, tdst_hbm, src_hbm, dst_hbm, abt_hbm, z_hbm,
          ex_hbm, acc_hbm, gs_v, gd_v, p_v, exb_v, srcv, dstv, abt_v, sh,
          ss0, ss1, sd0, sd1):
        c = lax.axis_index("c")
        s = lax.axis_index("s")
        wid = s * 2 + c
        pltpu.sync_copy(src_hbm.at[wid], srcv)
        pltpu.sync_copy(dst_hbm.at[wid], dstv)
        pltpu.sync_copy(abt_hbm, abt_v)

        @pl.when(s == 0)
        def _():
            pltpu.sync_copy(z_hbm, sh)

        plsc.subcore_barrier()

        sems_s = (ss0, ss1)
        sems_d = (sd0, sd1)

        lane = lax.iota(jnp.int32, 16)
        maskb = [lane == ee for ee in range(16)]
        a_chunks = [abt_v[j, :] for j in range(4)]

        def gather_pair(g, par):
            return (
                pltpu.make_async_copy(
                    tsrc_hbm.at[srcv.at[g]], gs_v.at[par], sems_s[par]),
                pltpu.make_async_copy(
                    tdst_hbm.at[dstv.at[g]], gd_v.at[par], sems_d[par]),
            )

        a0, b0 = gather_pair(0, 0)
        a0.start()
        b0.start()

        rows0 = lax.iota(jnp.int32, 16)
        base_w = wid * ew

        def process(g, buf):
            aw, bw = gather_pair(g, buf)
            aw.wait()
            bw.wait()

            @pl.when(g + 1 < nch)
            def _():
                an, bn = gather_pair(g + 1, 1 - buf)
                an.start()
                bn.start()

            gs = gs_v.at[buf]
            gd = gd_v.at[buf]

            @pl.loop(0, _C // 16)
            def _(gr):
                rows = rows0 + gr * 16
                rb = gr * 16
                svals = jnp.zeros((16,), jnp.float32)
                for ee in range(16):
                    r = rb + ee
                    t01 = (jnp.abs(gs[r, pl.ds(0, 16)] - gd[r, pl.ds(0, 16)])
                           * a_chunks[0]
                           + jnp.abs(gs[r, pl.ds(16, 16)] - gd[r, pl.ds(16, 16)])
                           * a_chunks[1])
                    t23 = (jnp.abs(gs[r, pl.ds(32, 16)] - gd[r, pl.ds(32, 16)])
                           * a_chunks[2]
                           + jnp.abs(gs[r, pl.ds(48, 16)] - gd[r, pl.ds(48, 16)])
                           * a_chunks[3])
                    s_e = jnp.sum(t01 + t23)
                    svals = jnp.where(maskb[ee], s_e, svals)
                hnu = plsc.load_gather(
                    gs, [rows, jnp.full((16,), 64, jnp.int32)])
                ex = jnp.maximum(jnp.exp(jnp.maximum(svals, 0.0) - hnu), 1e-30)
                exb_v[pl.ds(gr * 16, 16)] = ex
                for ee in range(16):
                    r = rb + ee
                    ex_e = jnp.sum(jnp.where(maskb[ee], ex, 0.0))
                    p_v[r, pl.ds(0, 16)] = ex_e * gd[r, pl.ds(64, 16)]
                    p_v[r, pl.ds(16, 16)] = ex_e * gd[r, pl.ds(80, 16)]
                plsc.store_scatter(
                    p_v, [rows, jnp.full((16,), 32, jnp.int32)], ex)

            pltpu.sync_copy(p_v, sh.at[srcv.at[g]], add=True)
            pltpu.sync_copy(exb_v, ex_hbm.at[pl.ds(base_w + g * _C, _C)])

        process(0, 0)

        @pl.loop(0, (nch - 1) // 2)
        def _(t):
            for par in (0, 1):
                process(1 + t * 2 + par, 1 - par)

        plsc.subcore_barrier()

        @pl.when(s == 0)
        def _():
            pltpu.sync_copy(sh, acc_hbm.at[c])

    z = jnp.zeros((n, 40), jnp.float32)
    return k(tsrc, tdst, src3, dst3, abt, z)


def _tc3_body(acc_ref, w2_ref, xw2_ref, rs8_ref):
    tot = acc_ref[0] + acc_ref[1]
    rs = tot[:, 32:33]
    x1 = jax.nn.relu(jnp.where(rs > 0, tot[:, :32] / rs, 0.0))
    xw2_ref[...] = lax.dot(x1, w2_ref[...], preferred_element_type=jnp.float32)
    rs8_ref[...] = jnp.broadcast_to(rs, rs8_ref.shape)


def _tc3(acc, W2, block_n=1000):
    n = acc.shape[1]
    return pl.pallas_call(
        _tc3_body,
        grid=(n // block_n,),
        in_specs=[
            pl.BlockSpec((2, block_n, 40), lambda i: (0, i, 0)),
            pl.BlockSpec((32, 16), lambda i: (0, 0)),
        ],
        out_specs=[
            pl.BlockSpec((block_n, 16), lambda i: (i, 0)),
            pl.BlockSpec((block_n, 8), lambda i: (i, 0)),
        ],
        out_shape=[
            jax.ShapeDtypeStruct((n, 16), jnp.float32),
            jax.ShapeDtypeStruct((n, 8), jnp.float32),
        ],
    )(acc, W2)


def _sc2(xw2, rs8, ex, src2, dst2, n, e):
    @functools.partial(
        pl.kernel,
        out_type=(
            jax.ShapeDtypeStruct((1, e), jnp.float32),
            jax.ShapeDtypeStruct((2, n, 16), jnp.float32),
        ),
        mesh=_MESH,
        scratch_types=[
            pltpu.VMEM((_W, 16), jnp.float32),
            pltpu.VMEM((_W, 8), jnp.float32),
            pltpu.VMEM((_W, 16), jnp.float32),
            pltpu.VMEM_SHARED((n, 16), jnp.float32),
        ],
        compiler_params=_PARAMS,
    )
    def k(xw2_hbm, rs8_hbm, ex_hbm, src_hbm, dst_hbm, z_hbm,
          adj_hbm, out_hbm, g2_v, rs_v, p2_v, sh):
        c = lax.axis_index("c")
        s = lax.axis_index("s")

        @pl.when(s == 0)
        def _():
            pltpu.sync_copy(z_hbm, sh)

        plsc.subcore_barrier()

        rows0 = lax.iota(jnp.int32, 16)

        def body(ex_v, src_v, dst_v, adj_v):
            pltpu.sync_copy(xw2_hbm.at[dst_v.at[0]], g2_v)
            pltpu.sync_copy(rs8_hbm.at[src_v.at[0]], rs_v)

            @pl.loop(0, _W // 16)
            def _(g):
                rows = rows0 + g * 16
                exv = ex_v[0, pl.ds(g * 16, 16)]
                rsv = plsc.load_gather(rs_v, [rows, jnp.full((16,), 0, jnp.int32)])
                adj = exv / rsv
                adj_v[0, pl.ds(g * 16, 16)] = adj
                for kk in range(16):
                    col = plsc.load_gather(
                        g2_v, [rows, jnp.full((16,), kk, jnp.int32)])
                    plsc.store_scatter(
                        p2_v, [rows, jnp.full((16,), kk, jnp.int32)], adj * col)

            pltpu.sync_copy(p2_v, sh.at[src_v.at[0]], add=True)

        pltpu.emit_pipeline(
            body,
            grid=(e // _W,),
            in_specs=[
                pl.BlockSpec((1, _W), lambda i: (0, i)),
                pl.BlockSpec((1, _W), lambda i: (0, i)),
                pl.BlockSpec((1, _W), lambda i: (0, i)),
            ],
            out_specs=[pl.BlockSpec((1, _W), lambda i: (0, i))],
            core_axis_name=("c", "s"),
            dimension_semantics=(pltpu.PARALLEL,),
        )(ex_hbm, src_hbm, dst_hbm, adj_hbm)

        plsc.subcore_barrier()

        @pl.when(s == 0)
        def _():
            pltpu.sync_copy(sh, out_hbm.at[c])

    z = jnp.zeros((n, 16), jnp.float32)
    return k(xw2, rs8, ex, src2, dst2, z)


def _tc5_body(p_ref, o_ref):
    o_ref[...] = p_ref[0] + p_ref[1]


def _tc5(parts, block_n=1000):
    n, d = parts.shape[1], parts.shape[2]
    return pl.pallas_call(
        _tc5_body,
        grid=(n // block_n,),
        in_specs=[pl.BlockSpec((2, block_n, d), lambda i: (0, i, 0))],
        out_specs=pl.BlockSpec((block_n, d), lambda i: (i, 0)),
        out_shape=jax.ShapeDtypeStruct((n, d), jnp.float32),
    )(parts)


def kernel(x, edge, num_nodes, W_gl, a, W1, W2):
    n = x.shape[0]
    e = edge.shape[1]
    src2 = edge[0:1]
    dst2 = edge[1:2]
    ew = e // 32
    src3 = edge[0].reshape(32, ew // _C, _C)
    dst3 = edge[1].reshape(32, ew // _C, _C)

    h, tdst, hn8, U, abt = _tc1(x, W_gl, W1, a)
    tsrc = _tc1b(h, hn8, U)
    ex, acc = _sc1(tsrc, tdst, src3, dst3, abt, n, e)
    xw2, rs8 = _tc3(acc, W2)
    adj, parts = _sc2(xw2, rs8, ex.reshape(1, e), src2, dst2, n, e)
    output = _tc5(parts)
    return (output, adj[0], h)


# speedup vs baseline: 2.2697x; 1.0779x over previous
import functools
import jax
import jax.numpy as jnp
from jax import lax
from jax.experimental import pallas as pl
from jax.experimental.pallas import tpu as pltpu
from jax.experimental.pallas import tpu_sc as plsc

_MESH = plsc.VectorSubcoreMesh(core_axis_name="c", subcore_axis_name="s")
_PARAMS = pltpu.CompilerParams(use_tc_tiling_on_sc=False,
                               needs_layout_passes=False)
_W = 400


def _tc1_body(x_ref, wgl_ref, w1_ref, a_ref, h_ref, tdst_ref, hn8_ref, u_ref,
              abt_ref):
    i = pl.program_id(0)
    x = x_ref[...]
    h = lax.dot(x, wgl_ref[...], preferred_element_type=jnp.float32)
    h_ref[...] = h
    tdst_ref[:, :64] = h
    tdst_ref[:, 64:] = lax.dot(x, w1_ref[...], preferred_element_type=jnp.float32)
    anorm = jnp.sqrt(jnp.sum(a_ref[...] ** 2))
    hn = anorm * jnp.sqrt(jnp.sum(h * h, axis=1, keepdims=True))
    hn8_ref[...] = jnp.broadcast_to(hn, hn8_ref.shape)
    bmax = jnp.max(hn).reshape(1, 1)
    u_ref[...] = jnp.where(i == 0, bmax, jnp.maximum(u_ref[...], bmax))
    abt_ref[...] = jnp.reshape(a_ref[...], abt_ref.shape)


def _tc1(x, W_gl, W1, a, block_n=1000):
    n, d = x.shape
    return pl.pallas_call(
        _tc1_body,
        grid=(n // block_n,),
        in_specs=[
            pl.BlockSpec((block_n, d), lambda i: (i, 0)),
            pl.BlockSpec((d, 64), lambda i: (0, 0)),
            pl.BlockSpec((d, 32), lambda i: (0, 0)),
            pl.BlockSpec((64, 1), lambda i: (0, 0)),
        ],
        out_specs=[
            pl.BlockSpec((block_n, 64), lambda i: (i, 0)),
            pl.BlockSpec((block_n, 96), lambda i: (i, 0)),
            pl.BlockSpec((block_n, 8), lambda i: (i, 0)),
            pl.BlockSpec((1, 1), lambda i: (0, 0)),
            pl.BlockSpec((4, 16), lambda i: (0, 0)),
        ],
        out_shape=[
            jax.ShapeDtypeStruct((n, 64), jnp.float32),
            jax.ShapeDtypeStruct((n, 96), jnp.float32),
            jax.ShapeDtypeStruct((n, 8), jnp.float32),
            jax.ShapeDtypeStruct((1, 1), jnp.float32),
            jax.ShapeDtypeStruct((4, 16), jnp.float32),
        ],
    )(x, W_gl, W1, a)


def _tc1b_body(h_ref, hn8_ref, u_ref, tsrc_ref):
    tsrc_ref[:, :64] = h_ref[...]
    tsrc_ref[:, 64:] = hn8_ref[...] + u_ref[0, 0]


def _tc1b(h, hn8, U, block_n=1000):
    n = h.shape[0]
    return pl.pallas_call(
        _tc1b_body,
        grid=(n // block_n,),
        in_specs=[
            pl.BlockSpec((block_n, 64), lambda i: (i, 0)),
            pl.BlockSpec((block_n, 8), lambda i: (i, 0)),
            pl.BlockSpec((1, 1), lambda i: (0, 0)),
        ],
        out_specs=pl.BlockSpec((block_n, 72), lambda i: (i, 0)),
        out_shape=jax.ShapeDtypeStruct((n, 72), jnp.float32),
    )(h, hn8, U)


_C = 80


def _sc1(tsrc, tdst, src3, dst3, abt, n, e):
    ew = e // 32
    nch = ew // _C

    @functools.partial(
        pl.kernel,
        out_type=(
            jax.ShapeDtypeStruct((e,), jnp.float32),
            jax.ShapeDtypeStruct((2, n, 40), jnp.float32),
        ),
        mesh=_MESH,
        scratch_types=[
            pltpu.VMEM((2, _C, 72), jnp.float32),
            pltpu.VMEM((2, _C, 96), jnp.float32),
            pltpu.VMEM((_C, 40), jnp.float32),
            pltpu.VMEM((_C,), jnp.float32),
            pltpu.VMEM((nch, _C), jnp.int32),
            pltpu.VMEM((nch, _C), jnp.int32),
            pltpu.VMEM((4, 16), jnp.float32),
            pltpu.VMEM_SHARED((n, 40), jnp.float32),
            pltpu.SemaphoreType.DMA,
            pltpu.SemaphoreType.DMA,
            pltpu.SemaphoreType.DMA,
            pltpu.SemaphoreType.DMA,
        ],
        compiler_params=_PARAMS,
    )
    def k(tsrc_hbm, tdst_hbm, src_hbm, dst_hbm, abt_hbm, z_hbm,
          ex_hbm, acc_hbm, gs_v, gd_v, p_v, exb_v, srcv, dstv, abt_v, sh,
          ss0, ss1, sd0, sd1):
        c = lax.axis_index("c")
        s = lax.axis_index("s")
        wid = s * 2 + c
        pltpu.sync_copy(src_hbm.at[wid], srcv)
        pltpu.sync_copy(dst_hbm.at[wid], dstv)
        pltpu.sync_copy(abt_hbm, abt_v)

        @pl.when(s == 0)
        def _():
            pltpu.sync_copy(z_hbm, sh)

        plsc.subcore_barrier()

        sems_s = (ss0, ss1)
        sems_d = (sd0, sd1)

        lane = lax.iota(jnp.int32, 16)
        maskb = [lane == ee for ee in range(16)]
        a_chunks = [abt_v[j, :] for j in range(4)]

        def gather_pair(g, par):
            return (
                pltpu.make_async_copy(
                    tsrc_hbm.at[srcv.at[g]], gs_v.at[par], sems_s[par]),
                pltpu.make_async_copy(
                    tdst_hbm.at[dstv.at[g]], gd_v.at[par], sems_d[par]),
            )

        a0, b0 = gather_pair(0, 0)
        a0.start()
        b0.start()

        rows0 = lax.iota(jnp.int32, 16)
        base_w = wid * ew

        def process(g, buf):
            aw, bw = gather_pair(g, buf)
            aw.wait()
            bw.wait()

            @pl.when(g + 1 < nch)
            def _():
                an, bn = gather_pair(g + 1, 1 - buf)
                an.start()
                bn.start()

            gs = gs_v.at[buf]
            gd = gd_v.at[buf]

            @pl.loop(0, _C // 16)
            def _(gr):
                rows = rows0 + gr * 16
                rb = gr * 16
                svals = jnp.zeros((16,), jnp.float32)
                for ee in range(16):
                    r = rb + ee
                    t01 = (jnp.abs(gs[r, pl.ds(0, 16)] - gd[r, pl.ds(0, 16)])
                           * a_chunks[0]
                           + jnp.abs(gs[r, pl.ds(16, 16)] - gd[r, pl.ds(16, 16)])
                           * a_chunks[1])
                    t23 = (jnp.abs(gs[r, pl.ds(32, 16)] - gd[r, pl.ds(32, 16)])
                           * a_chunks[2]
                           + jnp.abs(gs[r, pl.ds(48, 16)] - gd[r, pl.ds(48, 16)])
                           * a_chunks[3])
                    s_e = jnp.sum(t01 + t23)
                    svals = jnp.where(maskb[ee], s_e, svals)
                hnu = plsc.load_gather(
                    gs, [rows, jnp.full((16,), 64, jnp.int32)])
                ex = jnp.maximum(jnp.exp(jnp.maximum(svals, 0.0) - hnu), 1e-30)
                exb_v[pl.ds(gr * 16, 16)] = ex
                for ee in range(16):
                    r = rb + ee
                    ex_e = jnp.sum(jnp.where(maskb[ee], ex, 0.0))
                    p_v[r, pl.ds(0, 16)] = ex_e * gd[r, pl.ds(64, 16)]
                    p_v[r, pl.ds(16, 16)] = ex_e * gd[r, pl.ds(80, 16)]
                plsc.store_scatter(
                    p_v, [rows, jnp.full((16,), 32, jnp.int32)], ex)

            pltpu.sync_copy(p_v, sh.at[srcv.at[g]], add=True)
            pltpu.sync_copy(exb_v, ex_hbm.at[pl.ds(base_w + g * _C, _C)])

        process(0, 0)

        @pl.loop(0, (nch - 1) // 2)
        def _(t):
            for par in (0, 1):
                process(1 + t * 2 + par, 1 - par)

        plsc.subcore_barrier()

        @pl.when(s == 0)
        def _():
            pltpu.sync_copy(sh, acc_hbm.at[c])

    z = jnp.zeros((n, 40), jnp.float32)
    return k(tsrc, tdst, src3, dst3, abt, z)


def _tc3_body(acc_ref, w2_ref, xw2_ref, rs8_ref):
    tot = acc_ref[0] + acc_ref[1]
    rs = tot[:, 32:33]
    x1 = jax.nn.relu(jnp.where(rs > 0, tot[:, :32] / rs, 0.0))
    xw2_ref[...] = lax.dot(x1, w2_ref[...], preferred_element_type=jnp.float32)
    rs8_ref[...] = jnp.broadcast_to(rs, rs8_ref.shape)


def _tc3(acc, W2, block_n=1000):
    n = acc.shape[1]
    return pl.pallas_call(
        _tc3_body,
        grid=(n // block_n,),
        in_specs=[
            pl.BlockSpec((2, block_n, 40), lambda i: (0, i, 0)),
            pl.BlockSpec((32, 16), lambda i: (0, 0)),
        ],
        out_specs=[
            pl.BlockSpec((block_n, 16), lambda i: (i, 0)),
            pl.BlockSpec((block_n, 8), lambda i: (i, 0)),
        ],
        out_shape=[
            jax.ShapeDtypeStruct((n, 16), jnp.float32),
            jax.ShapeDtypeStruct((n, 8), jnp.float32),
        ],
    )(acc, W2)


def _sc2(xw2, rs8, ex, src2, dst2, n, e):
    @functools.partial(
        pl.kernel,
        out_type=(
            jax.ShapeDtypeStruct((1, e), jnp.float32),
            jax.ShapeDtypeStruct((2, n, 16), jnp.float32),
        ),
        mesh=_MESH,
        scratch_types=[
            pltpu.VMEM((_W, 16), jnp.float32),
            pltpu.VMEM((_W, 8), jnp.float32),
            pltpu.VMEM((_W, 16), jnp.float32),
            pltpu.VMEM_SHARED((n, 16), jnp.float32),
        ],
        compiler_params=_PARAMS,
    )
    def k(xw2_hbm, rs8_hbm, ex_hbm, src_hbm, dst_hbm, z_hbm,
          adj_hbm, out_hbm, g2_v, rs_v, p2_v, sh):
        c = lax.axis_index("c")
        s = lax.axis_index("s")

        @pl.when(s == 0)
        def _():
            pltpu.sync_copy(z_hbm, sh)

        plsc.subcore_barrier()

        rows0 = lax.iota(jnp.int32, 16)
        lane = lax.iota(jnp.int32, 16)
        maskb = [lane == ee for ee in range(16)]

        def body(ex_v, src_v, dst_v, adj_v):
            pltpu.sync_copy(xw2_hbm.at[dst_v.at[0]], g2_v)
            pltpu.sync_copy(rs8_hbm.at[src_v.at[0]], rs_v)

            @pl.loop(0, _W // 16)
            def _(g):
                rows = rows0 + g * 16
                rb = g * 16
                exv = ex_v[0, pl.ds(rb, 16)]
                rsv = plsc.load_gather(rs_v, [rows, jnp.full((16,), 0, jnp.int32)])
                adj = exv / rsv
                adj_v[0, pl.ds(rb, 16)] = adj
                for ee in range(16):
                    r = rb + ee
                    adj_e = jnp.sum(jnp.where(maskb[ee], adj, 0.0))
                    p2_v[r, pl.ds(0, 16)] = adj_e * g2_v[r, pl.ds(0, 16)]

            pltpu.sync_copy(p2_v, sh.at[src_v.at[0]], add=True)

        pltpu.emit_pipeline(
            body,
            grid=(e // _W,),
            in_specs=[
                pl.BlockSpec((1, _W), lambda i: (0, i)),
                pl.BlockSpec((1, _W), lambda i: (0, i)),
                pl.BlockSpec((1, _W), lambda i: (0, i)),
            ],
            out_specs=[pl.BlockSpec((1, _W), lambda i: (0, i))],
            core_axis_name=("c", "s"),
            dimension_semantics=(pltpu.PARALLEL,),
        )(ex_hbm, src_hbm, dst_hbm, adj_hbm)

        plsc.subcore_barrier()

        @pl.when(s == 0)
        def _():
            pltpu.sync_copy(sh, out_hbm.at[c])

    z = jnp.zeros((n, 16), jnp.float32)
    return k(xw2, rs8, ex, src2, dst2, z)


def _tc5_body(p_ref, o_ref):
    o_ref[...] = p_ref[0] + p_ref[1]


def _tc5(parts, block_n=1000):
    n, d = parts.shape[1], parts.shape[2]
    return pl.pallas_call(
        _tc5_body,
        grid=(n // block_n,),
        in_specs=[pl.BlockSpec((2, block_n, d), lambda i: (0, i, 0))],
        out_specs=pl.BlockSpec((block_n, d), lambda i: (i, 0)),
        out_shape=jax.ShapeDtypeStruct((n, d), jnp.float32),
    )(parts)


def kernel(x, edge, num_nodes, W_gl, a, W1, W2):
    n = x.shape[0]
    e = edge.shape[1]
    src2 = edge[0:1]
    dst2 = edge[1:2]
    ew = e // 32
    src3 = edge[0].reshape(32, ew // _C, _C)
    dst3 = edge[1].reshape(32, ew // _C, _C)

    h, tdst, hn8, U, abt = _tc1(x, W_gl, W1, a)
    tsrc = _tc1b(h, hn8, U)
    ex, acc = _sc1(tsrc, tdst, src3, dst3, abt, n, e)
    xw2, rs8 = _tc3(acc, W2)
    adj, parts = _sc2(xw2, rs8, ex.reshape(1, e), src2, dst2, n, e)
    output = _tc5(parts)
    return (output, adj[0], h)


# speedup vs baseline: 2.4062x; 1.0601x over previous
import functools
import jax
import jax.numpy as jnp
from jax import lax
from jax.experimental import pallas as pl
from jax.experimental.pallas import tpu as pltpu
from jax.experimental.pallas import tpu_sc as plsc

_MESH = plsc.VectorSubcoreMesh(core_axis_name="c", subcore_axis_name="s")
_PARAMS = pltpu.CompilerParams(use_tc_tiling_on_sc=False,
                               needs_layout_passes=False)
_W = 400


def _tc1_body(x_ref, wgl_ref, w1_ref, a_ref, h_ref, tdst_ref, hn8_ref, u_ref,
              abt_ref):
    i = pl.program_id(0)
    x = x_ref[...]
    h = lax.dot(x, wgl_ref[...], preferred_element_type=jnp.float32)
    h_ref[...] = h
    tdst_ref[:, :64] = h
    tdst_ref[:, 64:] = lax.dot(x, w1_ref[...], preferred_element_type=jnp.float32)
    anorm = jnp.sqrt(jnp.sum(a_ref[...] ** 2))
    hn = anorm * jnp.sqrt(jnp.sum(h * h, axis=1, keepdims=True))
    hn8_ref[...] = jnp.broadcast_to(hn, hn8_ref.shape)
    bmax = jnp.max(hn).reshape(1, 1)
    u_ref[...] = jnp.where(i == 0, bmax, jnp.maximum(u_ref[...], bmax))
    abt_ref[...] = jnp.reshape(a_ref[...], abt_ref.shape)


def _tc1(x, W_gl, W1, a, block_n=1000):
    n, d = x.shape
    return pl.pallas_call(
        _tc1_body,
        grid=(n // block_n,),
        in_specs=[
            pl.BlockSpec((block_n, d), lambda i: (i, 0)),
            pl.BlockSpec((d, 64), lambda i: (0, 0)),
            pl.BlockSpec((d, 32), lambda i: (0, 0)),
            pl.BlockSpec((64, 1), lambda i: (0, 0)),
        ],
        out_specs=[
            pl.BlockSpec((block_n, 64), lambda i: (i, 0)),
            pl.BlockSpec((block_n, 96), lambda i: (i, 0)),
            pl.BlockSpec((block_n, 8), lambda i: (i, 0)),
            pl.BlockSpec((1, 1), lambda i: (0, 0)),
            pl.BlockSpec((4, 16), lambda i: (0, 0)),
        ],
        out_shape=[
            jax.ShapeDtypeStruct((n, 64), jnp.float32),
            jax.ShapeDtypeStruct((n, 96), jnp.float32),
            jax.ShapeDtypeStruct((n, 8), jnp.float32),
            jax.ShapeDtypeStruct((1, 1), jnp.float32),
            jax.ShapeDtypeStruct((4, 16), jnp.float32),
        ],
    )(x, W_gl, W1, a)


def _tc1b_body(h_ref, hn8_ref, u_ref, tsrc_ref):
    tsrc_ref[:, :64] = h_ref[...]
    tsrc_ref[:, 64:] = hn8_ref[...] + u_ref[0, 0]


def _tc1b(h, hn8, U, block_n=1000):
    n = h.shape[0]
    return pl.pallas_call(
        _tc1b_body,
        grid=(n // block_n,),
        in_specs=[
            pl.BlockSpec((block_n, 64), lambda i: (i, 0)),
            pl.BlockSpec((block_n, 8), lambda i: (i, 0)),
            pl.BlockSpec((1, 1), lambda i: (0, 0)),
        ],
        out_specs=pl.BlockSpec((block_n, 72), lambda i: (i, 0)),
        out_shape=jax.ShapeDtypeStruct((n, 72), jnp.float32),
    )(h, hn8, U)


_C = 80


def _sc1(tsrc, tdst, src3, dst3, abt, n, e):
    ew = e // 32
    nch = ew // _C

    @functools.partial(
        pl.kernel,
        out_type=(
            jax.ShapeDtypeStruct((e,), jnp.float32),
            jax.ShapeDtypeStruct((2, n, 40), jnp.float32),
        ),
        mesh=_MESH,
        scratch_types=[
            pltpu.VMEM((2, _C, 72), jnp.float32),
            pltpu.VMEM((2, _C, 96), jnp.float32),
            pltpu.VMEM((_C, 40), jnp.float32),
            pltpu.VMEM((_C,), jnp.float32),
            pltpu.VMEM((nch, _C), jnp.int32),
            pltpu.VMEM((nch, _C), jnp.int32),
            pltpu.VMEM((4, 16), jnp.float32),
            pltpu.VMEM_SHARED((n, 40), jnp.float32),
            pltpu.SemaphoreType.DMA,
            pltpu.SemaphoreType.DMA,
            pltpu.SemaphoreType.DMA,
            pltpu.SemaphoreType.DMA,
        ],
        compiler_params=_PARAMS,
    )
    def k(tsrc_hbm, tdst_hbm, src_hbm, dst_hbm, abt_hbm, z_hbm,
          ex_hbm, acc_hbm, gs_v, gd_v, p_v, exb_v, srcv, dstv, abt_v, sh,
          ss0, ss1, sd0, sd1):
        c = lax.axis_index("c")
        s = lax.axis_index("s")
        wid = s * 2 + c
        pltpu.sync_copy(src_hbm.at[wid], srcv)
        pltpu.sync_copy(dst_hbm.at[wid], dstv)
        pltpu.sync_copy(abt_hbm, abt_v)

        @pl.when(s == 0)
        def _():
            pltpu.sync_copy(z_hbm, sh)

        plsc.subcore_barrier()

        sems_s = (ss0, ss1)
        sems_d = (sd0, sd1)

        lane = lax.iota(jnp.int32, 16)
        maskb = [lane == ee for ee in range(16)]
        a_chunks = [abt_v[j, :] for j in range(4)]

        def gather_pair(g, par):
            return (
                pltpu.make_async_copy(
                    tsrc_hbm.at[srcv.at[g]], gs_v.at[par], sems_s[par]),
                pltpu.make_async_copy(
                    tdst_hbm.at[dstv.at[g]], gd_v.at[par], sems_d[par]),
            )

        a0, b0 = gather_pair(0, 0)
        a0.start()
        b0.start()

        rows0 = lax.iota(jnp.int32, 16)
        base_w = wid * ew

        def process(g, buf):
            aw, bw = gather_pair(g, buf)
            aw.wait()
            bw.wait()

            @pl.when(g + 1 < nch)
            def _():
                an, bn = gather_pair(g + 1, 1 - buf)
                an.start()
                bn.start()

            gs = gs_v.at[buf]
            gd = gd_v.at[buf]

            @pl.loop(0, _C // 16)
            def _(gr):
                rows = rows0 + gr * 16
                rb = gr * 16
                svals = jnp.zeros((16,), jnp.float32)
                for ee in range(16):
                    r = rb + ee
                    t01 = (jnp.abs(gs[r, pl.ds(0, 16)] - gd[r, pl.ds(0, 16)])
                           * a_chunks[0]
                           + jnp.abs(gs[r, pl.ds(16, 16)] - gd[r, pl.ds(16, 16)])
                           * a_chunks[1])
                    t23 = (jnp.abs(gs[r, pl.ds(32, 16)] - gd[r, pl.ds(32, 16)])
                           * a_chunks[2]
                           + jnp.abs(gs[r, pl.ds(48, 16)] - gd[r, pl.ds(48, 16)])
                           * a_chunks[3])
                    s_e = jnp.sum(t01 + t23)
                    svals = jnp.where(maskb[ee], s_e, svals)
                hnu = plsc.load_gather(
                    gs, [rows, jnp.full((16,), 64, jnp.int32)])
                ex = jnp.maximum(jnp.exp(jnp.maximum(svals, 0.0) - hnu), 1e-30)
                exb_v[pl.ds(gr * 16, 16)] = ex
                for ee in range(16):
                    r = rb + ee
                    ex_e = jnp.sum(jnp.where(maskb[ee], ex, 0.0))
                    p_v[r, pl.ds(0, 16)] = ex_e * gd[r, pl.ds(64, 16)]
                    p_v[r, pl.ds(16, 16)] = ex_e * gd[r, pl.ds(80, 16)]
                plsc.store_scatter(
                    p_v, [rows, jnp.full((16,), 32, jnp.int32)], ex)

            pltpu.sync_copy(p_v, sh.at[srcv.at[g]], add=True)
            pltpu.sync_copy(exb_v, ex_hbm.at[pl.ds(base_w + g * _C, _C)])

        process(0, 0)

        @pl.loop(0, (nch - 1) // 2)
        def _(t):
            for par in (0, 1):
                process(1 + t * 2 + par, 1 - par)

        plsc.subcore_barrier()

        @pl.when(s == 0)
        def _():
            pltpu.sync_copy(sh, acc_hbm.at[c])

    z = jnp.zeros((n, 40), jnp.float32)
    return k(tsrc, tdst, src3, dst3, abt, z)


def _tc3_body(acc_ref, w2_ref, xw2_ref, rs8_ref):
    tot = acc_ref[0] + acc_ref[1]
    rs = tot[:, 32:33]
    x1 = jax.nn.relu(jnp.where(rs > 0, tot[:, :32] / rs, 0.0))
    xw2_ref[...] = lax.dot(x1, w2_ref[...], preferred_element_type=jnp.float32)
    rs8_ref[...] = jnp.broadcast_to(rs, rs8_ref.shape)


def _tc3(acc, W2, block_n=1000):
    n = acc.shape[1]
    return pl.pallas_call(
        _tc3_body,
        grid=(n // block_n,),
        in_specs=[
            pl.BlockSpec((2, block_n, 40), lambda i: (0, i, 0)),
            pl.BlockSpec((32, 16), lambda i: (0, 0)),
        ],
        out_specs=[
            pl.BlockSpec((block_n, 16), lambda i: (i, 0)),
            pl.BlockSpec((block_n, 8), lambda i: (i, 0)),
        ],
        out_shape=[
            jax.ShapeDtypeStruct((n, 16), jnp.float32),
            jax.ShapeDtypeStruct((n, 8), jnp.float32),
        ],
    )(acc, W2)


def _sc2(xw2, rs8, ex, src3, dst3, n, e):
    ew = e // 32
    nch = ew // _C

    @functools.partial(
        pl.kernel,
        out_type=(
            jax.ShapeDtypeStruct((e,), jnp.float32),
            jax.ShapeDtypeStruct((2, n, 16), jnp.float32),
        ),
        mesh=_MESH,
        scratch_types=[
            pltpu.VMEM((2, _C, 16), jnp.float32),
            pltpu.VMEM((2, _C, 8), jnp.float32),
            pltpu.VMEM((_C, 16), jnp.float32),
            pltpu.VMEM((_C,), jnp.float32),
            pltpu.VMEM((ew,), jnp.float32),
            pltpu.VMEM((nch, _C), jnp.int32),
            pltpu.VMEM((nch, _C), jnp.int32),
            pltpu.VMEM_SHARED((n, 16), jnp.float32),
            pltpu.SemaphoreType.DMA,
            pltpu.SemaphoreType.DMA,
            pltpu.SemaphoreType.DMA,
            pltpu.SemaphoreType.DMA,
        ],
        compiler_params=_PARAMS,
    )
    def k(xw2_hbm, rs8_hbm, ex_hbm, src_hbm, dst_hbm, z_hbm,
          adj_hbm, out_hbm, g2_v, rs_v, p2_v, adjb_v, exv_all, srcv, dstv,
          sh, ss0, ss1, sd0, sd1):
        c = lax.axis_index("c")
        s = lax.axis_index("s")
        wid = s * 2 + c
        base_w = wid * ew
        pltpu.sync_copy(src_hbm.at[wid], srcv)
        pltpu.sync_copy(dst_hbm.at[wid], dstv)
        pltpu.sync_copy(ex_hbm.at[pl.ds(base_w, ew)], exv_all)

        @pl.when(s == 0)
        def _():
            pltpu.sync_copy(z_hbm, sh)

        plsc.subcore_barrier()

        sems_s = (ss0, ss1)
        sems_d = (sd0, sd1)
        rows0 = lax.iota(jnp.int32, 16)
        lane = lax.iota(jnp.int32, 16)
        maskb = [lane == ee for ee in range(16)]

        def gather_pair(g, par):
            return (
                pltpu.make_async_copy(
                    rs8_hbm.at[srcv.at[g]], rs_v.at[par], sems_s[par]),
                pltpu.make_async_copy(
                    xw2_hbm.at[dstv.at[g]], g2_v.at[par], sems_d[par]),
            )

        a0, b0 = gather_pair(0, 0)
        a0.start()
        b0.start()

        def process(g, buf):
            aw, bw = gather_pair(g, buf)
            aw.wait()
            bw.wait()

            @pl.when(g + 1 < nch)
            def _():
                an, bn = gather_pair(g + 1, 1 - buf)
                an.start()
                bn.start()

            g2 = g2_v.at[buf]
            rs = rs_v.at[buf]

            @pl.loop(0, _C // 16)
            def _(gr):
                rows = rows0 + gr * 16
                rb = gr * 16
                exv = exv_all[pl.ds(g * _C + rb, 16)]
                rsv = plsc.load_gather(rs, [rows, jnp.full((16,), 0, jnp.int32)])
                adj = exv / rsv
                adjb_v[pl.ds(rb, 16)] = adj
                for ee in range(16):
                    r = rb + ee
                    adj_e = jnp.sum(jnp.where(maskb[ee], adj, 0.0))
                    p2_v[r, pl.ds(0, 16)] = adj_e * g2[r, pl.ds(0, 16)]

            pltpu.sync_copy(p2_v, sh.at[srcv.at[g]], add=True)
            pltpu.sync_copy(adjb_v, adj_hbm.at[pl.ds(base_w + g * _C, _C)])

        process(0, 0)

        @pl.loop(0, (nch - 1) // 2)
        def _(t):
            for par in (0, 1):
                process(1 + t * 2 + par, 1 - par)

        plsc.subcore_barrier()

        @pl.when(s == 0)
        def _():
            pltpu.sync_copy(sh, out_hbm.at[c])

    z = jnp.zeros((n, 16), jnp.float32)
    return k(xw2, rs8, ex, src3, dst3, z)


def _tc5_body(p_ref, o_ref):
    o_ref[...] = p_ref[0] + p_ref[1]


def _tc5(parts, block_n=1000):
    n, d = parts.shape[1], parts.shape[2]
    return pl.pallas_call(
        _tc5_body,
        grid=(n // block_n,),
        in_specs=[pl.BlockSpec((2, block_n, d), lambda i: (0, i, 0))],
        out_specs=pl.BlockSpec((block_n, d), lambda i: (i, 0)),
        out_shape=jax.ShapeDtypeStruct((n, d), jnp.float32),
    )(parts)


def kernel(x, edge, num_nodes, W_gl, a, W1, W2):
    n = x.shape[0]
    e = edge.shape[1]
    src2 = edge[0:1]
    dst2 = edge[1:2]
    ew = e // 32
    src3 = edge[0].reshape(32, ew // _C, _C)
    dst3 = edge[1].reshape(32, ew // _C, _C)

    h, tdst, hn8, U, abt = _tc1(x, W_gl, W1, a)
    tsrc = _tc1b(h, hn8, U)
    ex, acc = _sc1(tsrc, tdst, src3, dst3, abt, n, e)
    xw2, rs8 = _tc3(acc, W2)
    adj, parts = _sc2(xw2, rs8, ex, src3, dst3, n, e)
    output = _tc5(parts)
    return (output, adj, h)
